# baseline (speedup 1.0000x reference)
import functools

import jax
import jax.numpy as jnp
from jax import lax
from jax.experimental import pallas as pl
from jax.experimental.pallas import tpu as pltpu
from jax.experimental.pallas import tpu_sc as plsc

_D = 64
_IW = 128
_SUB = 4
_CHUNK = _IW * _SUB


def _make_gather(total_rows: int):
    info = plsc.get_sparse_core_info()
    nw = 1 * info.num_subcores
    rows_per_w = total_rows // nw
    nchunk = rows_per_w // _CHUNK
    npair = nchunk // 2
    idx_rows = rows_per_w // _IW
    mesh = plsc.VectorSubcoreMesh(
        core_axis_name="c", subcore_axis_name="s", num_cores=1
    )

    @functools.partial(
        pl.kernel,
        mesh=mesh,
        out_type=jax.ShapeDtypeStruct((total_rows, _D), jnp.float32),
        scratch_types=[
            pltpu.VMEM((idx_rows, _IW), jnp.int32),
            pltpu.VMEM((_CHUNK, _D), jnp.float32),
            pltpu.VMEM((_CHUNK, _D), jnp.float32),
            pltpu.SemaphoreType.DMA,
            pltpu.SemaphoreType.DMA,
        ],
        compiler_params=pltpu.CompilerParams(
            use_tc_tiling_on_sc=False, skip_device_barrier=True
        ),
    )
    def gather(table_hbm, idx_hbm, out_hbm, idx_v, rows0, rows1, sg0, sg1):
        wid = lax.axis_index("s") + lax.axis_index("c")
        base = wid * rows_per_w

        pltpu.sync_copy(
            idx_hbm.at[pl.ds(pl.multiple_of(wid * idx_rows, 8), idx_rows)], idx_v
        )

        def fire(c, rows_v, sem):
            for j in range(_SUB):
                pltpu.async_copy(
                    table_hbm.at[idx_v.at[c * _SUB + j]],
                    rows_v.at[pl.ds(j * _IW, _IW)],
                    sem,
                )

        def drain(c, rows_v, sem):
            for j in range(_SUB):
                pltpu.make_async_copy(
                    table_hbm.at[idx_v.at[c * _SUB + j]],
                    rows_v.at[pl.ds(j * _IW, _IW)],
                    sem,
                ).wait()

        def wback(c, rows_v):
            off = pl.multiple_of(base + c * _CHUNK, _CHUNK)
            pltpu.sync_copy(rows_v, out_hbm.at[pl.ds(off, _CHUNK)])

        fire(0, rows0, sg0)

        def pair_body(p, carry):
            c0 = p * 2
            fire(c0 + 1, rows1, sg1)
            drain(c0, rows0, sg0)
            wback(c0, rows0)

            @pl.when(p + 1 < npair)
            def _():
                fire(c0 + 2, rows0, sg0)

            drain(c0 + 1, rows1, sg1)
            wback(c0 + 1, rows1)
            return carry

        lax.fori_loop(0, npair, pair_body, 0)

    return gather


def kernel(memory, t_layer, idx, contrast_idx):
    t, cap, d = memory.shape
    b = idx.shape[0]
    kp1 = contrast_idx.shape[1] + 1
    total = b * kp1
    table = jax.lax.dynamic_slice_in_dim(memory, t_layer, 1, axis=0).reshape(cap, d)
    full_idx = jnp.concatenate(
        [idx.astype(jnp.int32)[:, None], contrast_idx.astype(jnp.int32)], axis=1
    )
    idx2d = full_idx.reshape(total // _IW, _IW)
    half = total // 2
    g = _make_gather(half)
    o0 = g(table, idx2d[: idx2d.shape[0] // 2])
    o1 = g(table, idx2d[idx2d.shape[0] // 2 :])
    out = jnp.concatenate([o0, o1], axis=0)
    return out.reshape(b, kp1, d)


# speedup vs baseline: 1.1482x; 1.1482x over previous
import functools

import jax
import jax.numpy as jnp
from jax import lax
from jax.experimental import pallas as pl
from jax.experimental.pallas import tpu as pltpu
from jax.experimental.pallas import tpu_sc as plsc

_D = 64
_IW = 128
_SUB = 4
_CHUNK = _IW * _SUB


def _make_gather(total_rows: int):
    info = plsc.get_sparse_core_info()
    nw = info.num_cores * info.num_subcores
    rows_per_w = total_rows // nw
    nchunk = rows_per_w // _CHUNK
    npair = nchunk // 2
    idx_rows = rows_per_w // _IW
    mesh = plsc.VectorSubcoreMesh(core_axis_name="c", subcore_axis_name="s")

    @functools.partial(
        pl.kernel,
        mesh=mesh,
        out_type=jax.ShapeDtypeStruct((total_rows, _D), jnp.float32),
        scratch_types=[
            pltpu.VMEM((idx_rows, _IW), jnp.int32),
            pltpu.VMEM((_CHUNK, _D), jnp.float32),
            pltpu.VMEM((_CHUNK, _D), jnp.float32),
            pltpu.SemaphoreType.DMA,
            pltpu.SemaphoreType.DMA,
        ],
        compiler_params=pltpu.CompilerParams(
            use_tc_tiling_on_sc=False, skip_device_barrier=True
        ),
    )
    def gather(table_hbm, idx_hbm, out_hbm, idx_v, rows0, rows1, sg0, sg1):
        wid = lax.axis_index("s") * info.num_cores + lax.axis_index("c")
        base = wid * rows_per_w

        pltpu.sync_copy(
            idx_hbm.at[pl.ds(pl.multiple_of(wid * idx_rows, 8), idx_rows)], idx_v
        )

        def fire(c, rows_v, sem):
            for j in range(_SUB):
                pltpu.async_copy(
                    table_hbm.at[idx_v.at[c * _SUB + j]],
                    rows_v.at[pl.ds(j * _IW, _IW)],
                    sem,
                )

        def drain(c, rows_v, sem):
            for j in range(_SUB):
                pltpu.make_async_copy(
                    table_hbm.at[idx_v.at[c * _SUB + j]],
                    rows_v.at[pl.ds(j * _IW, _IW)],
                    sem,
                ).wait()

        def wback(c, rows_v):
            off = pl.multiple_of(base + c * _CHUNK, _CHUNK)
            pltpu.sync_copy(rows_v, out_hbm.at[pl.ds(off, _CHUNK)])

        fire(0, rows0, sg0)

        def pair_body(p, carry):
            c0 = p * 2
            fire(c0 + 1, rows1, sg1)
            drain(c0, rows0, sg0)
            wback(c0, rows0)

            @pl.when(p + 1 < npair)
            def _():
                fire(c0 + 2, rows0, sg0)

            drain(c0 + 1, rows1, sg1)
            wback(c0 + 1, rows1)
            return carry

        lax.fori_loop(0, npair, pair_body, 0)

    return gather


def kernel(memory, t_layer, idx, contrast_idx):
    t, cap, d = memory.shape
    b = idx.shape[0]
    kp1 = contrast_idx.shape[1] + 1
    total = b * kp1
    table = jax.lax.dynamic_slice_in_dim(memory, t_layer, 1, axis=0).reshape(cap, d)
    full_idx = jnp.concatenate(
        [idx.astype(jnp.int32)[:, None], contrast_idx.astype(jnp.int32)], axis=1
    )
    idx2d = full_idx.reshape(total // _IW, _IW)
    out = _make_gather(total)(table, idx2d)
    return out.reshape(b, kp1, d)


# speedup vs baseline: 1.1499x; 1.0015x over previous
import functools

import jax
import jax.numpy as jnp
from jax import lax
from jax.experimental import pallas as pl
from jax.experimental.pallas import tpu as pltpu
from jax.experimental.pallas import tpu_sc as plsc

_D = 64
_IW = 128
_SUB = 4
_CHUNK = _IW * _SUB


def _make_gather(total_rows: int):
    info = plsc.get_sparse_core_info()
    nw = info.num_cores * info.num_subcores
    rows_per_w = total_rows // nw
    nchunk = rows_per_w // _CHUNK
    npair = nchunk // 2
    idx_rows = rows_per_w // _IW
    mesh = plsc.VectorSubcoreMesh(core_axis_name="c", subcore_axis_name="s")

    @functools.partial(
        pl.kernel,
        mesh=mesh,
        out_type=jax.ShapeDtypeStruct((total_rows, _D), jnp.float32),
        scratch_types=[
            pltpu.VMEM((idx_rows, _IW), jnp.int32),
            pltpu.VMEM((_CHUNK, _D), jnp.float32),
            pltpu.VMEM((_CHUNK, _D), jnp.float32),
            pltpu.SemaphoreType.DMA,
            pltpu.SemaphoreType.DMA,
        ],
        compiler_params=pltpu.CompilerParams(use_tc_tiling_on_sc=False),
    )
    def gather(table_hbm, idx_hbm, out_hbm, idx_v, rows0, rows1, sg0, sg1):
        wid = lax.axis_index("s") * info.num_cores + lax.axis_index("c")
        base = wid * rows_per_w

        pltpu.sync_copy(
            idx_hbm.at[pl.ds(pl.multiple_of(wid * idx_rows, 8), idx_rows)], idx_v
        )

        def fire(c, rows_v, sem):
            for j in range(_SUB):
                pltpu.async_copy(
                    table_hbm.at[idx_v.at[c * _SUB + j]],
                    rows_v.at[pl.ds(j * _IW, _IW)],
                    sem,
                )

        def drain(c, rows_v, sem):
            for j in range(_SUB):
                pltpu.make_async_copy(
                    table_hbm.at[idx_v.at[c * _SUB + j]],
                    rows_v.at[pl.ds(j * _IW, _IW)],
                    sem,
                ).wait()

        def wback(c, rows_v):
            off = pl.multiple_of(base + c * _CHUNK, _CHUNK)
            pltpu.sync_copy(rows_v, out_hbm.at[pl.ds(off, _CHUNK)])

        fire(0, rows0, sg0)

        def pair_body(p, carry):
            c0 = p * 2
            fire(c0 + 1, rows1, sg1)
            drain(c0, rows0, sg0)
            wback(c0, rows0)

            @pl.when(p + 1 < npair)
            def _():
                fire(c0 + 2, rows0, sg0)

            drain(c0 + 1, rows1, sg1)
            wback(c0 + 1, rows1)
            return carry

        lax.fori_loop(0, npair, pair_body, 0)

    return gather


def kernel(memory, t_layer, idx, contrast_idx):
    t, cap, d = memory.shape
    b = idx.shape[0]
    kp1 = contrast_idx.shape[1] + 1
    total = b * kp1
    table = jax.lax.dynamic_slice_in_dim(memory, t_layer, 1, axis=0).reshape(cap, d)
    full_idx = jnp.concatenate(
        [idx.astype(jnp.int32)[:, None], contrast_idx.astype(jnp.int32)], axis=1
    )
    idx2d = full_idx.reshape(total // _IW, _IW)
    out = _make_gather(total)(table, idx2d)
    return out.reshape(b, kp1, d)
